# baseline (speedup 1.0000x reference)
import functools

import jax
import jax.numpy as jnp
from jax import lax
from jax.experimental import pallas as pl
from jax.experimental.pallas import tpu as pltpu
from jax.experimental.pallas import tpu_sc as plsc

_B = 16384
_NCAT = 26
_NCONT = 13
_D = 64
_ROWS = _B * _NCAT
_CHUNK = 128
_NC = 2
_NS = 16
_NW = _NC * _NS
_CPW = _ROWS // (_CHUNK * _NW)

_mesh = plsc.VectorSubcoreMesh(core_axis_name="c", subcore_axis_name="s")


@functools.partial(
    pl.kernel,
    out_type=jax.ShapeDtypeStruct((_ROWS, _D), jnp.float32),
    mesh=_mesh,
    compiler_params=pltpu.CompilerParams(use_tc_tiling_on_sc=False),
    scratch_types=[
        pltpu.VMEM((_CPW, _CHUNK), jnp.int32),
        pltpu.VMEM((_CHUNK, _D), jnp.float32),
        pltpu.SemaphoreType.DMA,
    ],
)
def _sc_gather(table_hbm, idx_hbm, out_hbm, idx_v, buf, sem):
    wid = lax.axis_index("s") * _NC + lax.axis_index("c")
    base_chunk = wid * _CPW
    pltpu.sync_copy(idx_hbm.at[pl.ds(base_chunk, _CPW)], idx_v)

    def body(j, carry):
        pltpu.async_copy(table_hbm.at[idx_v.at[j]], buf, sem).wait()
        pltpu.sync_copy(buf, out_hbm.at[pl.ds((base_chunk + j) * _CHUNK, _CHUNK)])
        return carry

    lax.fori_loop(0, _CPW, body, 0)


def _cont_body(x_ref, w_ref, b_ref, o_ref):
    o_ref[...] = (
        jnp.dot(x_ref[...], w_ref[...], preferred_element_type=jnp.float32,
                precision=jax.lax.Precision.HIGHEST)
        + b_ref[...]
    )


_BB = 2048
_DF = _NCONT * _D


def _cont_embed(xc, w2, b2):
    return pl.pallas_call(
        _cont_body,
        out_shape=jax.ShapeDtypeStruct((_B, _DF), jnp.float32),
        grid=(_B // _BB,),
        in_specs=[
            pl.BlockSpec((_BB, _NCONT), lambda i: (i, 0)),
            pl.BlockSpec((_NCONT, _DF), lambda i: (0, 0)),
            pl.BlockSpec((1, _DF), lambda i: (0, 0)),
        ],
        out_specs=pl.BlockSpec((_BB, _DF), lambda i: (i, 0)),
    )(xc, w2, b2)


def kernel(X, table, cont_w, cont_b):
    idx = X[:, :_NCAT].astype(jnp.int32).reshape(_NW * _CPW, _CHUNK)
    xc = X[:, _NCAT:_NCAT + _NCONT]
    w2 = (jnp.eye(_NCONT, dtype=jnp.float32)[:, :, None] * cont_w[None, :, :]
          ).reshape(_NCONT, _DF)
    b2 = cont_b.reshape(1, _DF)
    cat_flat = _sc_gather(table, idx)
    x_cont = _cont_embed(xc, w2, b2)
    return cat_flat.reshape(_B, _NCAT, _D), x_cont.reshape(_B, _NCONT, _D)


# speedup vs baseline: 1.0708x; 1.0708x over previous
import functools

import jax
import jax.numpy as jnp
from jax import lax
from jax.experimental import pallas as pl
from jax.experimental.pallas import tpu as pltpu
from jax.experimental.pallas import tpu_sc as plsc

_B = 16384
_NCAT = 26
_NCONT = 13
_D = 64
_ROWS = _B * _NCAT
_CHUNK = 128
_NC = 2
_NS = 16
_NW = _NC * _NS
_CPW = _ROWS // (_CHUNK * _NW)

_mesh = plsc.VectorSubcoreMesh(core_axis_name="c", subcore_axis_name="s")


_NBUF = 4
_DLY = 2
_NGRP = _CPW // _NBUF


@functools.partial(
    pl.kernel,
    out_type=jax.ShapeDtypeStruct((_ROWS, _D), jnp.float32),
    mesh=_mesh,
    compiler_params=pltpu.CompilerParams(use_tc_tiling_on_sc=False),
    scratch_types=[
        pltpu.VMEM((_CPW, _CHUNK), jnp.int32),
        [pltpu.VMEM((_CHUNK, _D), jnp.float32)] * _NBUF,
        [pltpu.SemaphoreType.DMA] * _NBUF,
        [pltpu.SemaphoreType.DMA] * _NBUF,
    ],
)
def _sc_gather(table_hbm, idx_hbm, out_hbm, idx_v, bufs, gsems, wsems):
    wid = lax.axis_index("s") * _NC + lax.axis_index("c")
    base_chunk = wid * _CPW
    pltpu.sync_copy(idx_hbm.at[pl.ds(base_chunk, _CPW)], idx_v)

    def gather(j, b):
        pltpu.async_copy(table_hbm.at[idx_v.at[j]], bufs[b], gsems[b])

    def drain_gather(b):
        pltpu.make_async_copy(table_hbm.at[idx_v.at[0]], bufs[b], gsems[b]).wait()

    def write(j, b):
        pltpu.async_copy(
            bufs[b], out_hbm.at[pl.ds((base_chunk + j) * _CHUNK, _CHUNK)], wsems[b])

    def drain_write(b):
        pltpu.make_async_copy(
            bufs[b], out_hbm.at[pl.ds(base_chunk * _CHUNK, _CHUNK)], wsems[b]).wait()

    for b in range(_NBUF):
        gather(b, b)
    for b in range(_DLY, _NBUF):
        drain_gather(b - _DLY)
        write(b - _DLY, b - _DLY)

    def body(g, carry):
        j0 = g * _NBUF
        for b in range(_NBUF):
            drain_write(b)
            gather(j0 + b, b)
            bp = (b - _DLY) % _NBUF
            drain_gather(bp)
            write(j0 + b - _DLY, bp)
        return carry

    lax.fori_loop(1, _NGRP, body, 0)

    jlast = _NGRP * _NBUF
    for j in range(jlast, jlast + _DLY):
        bp = (j - _DLY) % _NBUF
        drain_gather(bp)
        write(j - _DLY, bp)
    for b in range(_NBUF):
        drain_write(b)


def _cont_body(x_ref, w_ref, b_ref, o_ref):
    o_ref[...] = (
        jnp.dot(x_ref[...], w_ref[...], preferred_element_type=jnp.float32,
                precision=jax.lax.Precision.HIGHEST)
        + b_ref[...]
    )


_BB = 2048
_DF = _NCONT * _D


def _cont_embed(xc, w2, b2):
    return pl.pallas_call(
        _cont_body,
        out_shape=jax.ShapeDtypeStruct((_B, _DF), jnp.float32),
        grid=(_B // _BB,),
        in_specs=[
            pl.BlockSpec((_BB, _NCONT), lambda i: (i, 0)),
            pl.BlockSpec((_NCONT, _DF), lambda i: (0, 0)),
            pl.BlockSpec((1, _DF), lambda i: (0, 0)),
        ],
        out_specs=pl.BlockSpec((_BB, _DF), lambda i: (i, 0)),
    )(xc, w2, b2)


def kernel(X, table, cont_w, cont_b):
    idx = X[:, :_NCAT].astype(jnp.int32).reshape(_NW * _CPW, _CHUNK)
    xc = X[:, _NCAT:_NCAT + _NCONT]
    w2 = (jnp.eye(_NCONT, dtype=jnp.float32)[:, :, None] * cont_w[None, :, :]
          ).reshape(_NCONT, _DF)
    b2 = cont_b.reshape(1, _DF)
    cat_flat = _sc_gather(table, idx)
    x_cont = _cont_embed(xc, w2, b2)
    return cat_flat.reshape(_B, _NCAT, _D), x_cont.reshape(_B, _NCONT, _D)
